# Initial kernel scaffold; baseline (speedup 1.0000x reference)
import jax
import jax.numpy as jnp
from jax.experimental import pallas as pl


def kernel(user_idx, item_idx, user_hist, item_hist, user_embed_w, item_embed_w, W_u, bW_u, v_u, bv_u, norm_u_g, norm_u_b, norm_i_g, norm_i_b, pred_W, pred_b):
    raise NotImplementedError("write your pallas kernel here")


# speedup vs baseline: 2.6854x; 2.6854x over previous
import functools

import jax
import jax.numpy as jnp
from jax import lax
from jax.experimental import pallas as pl
from jax.experimental.pallas import tpu as pltpu
from jax.experimental.pallas import tpu_sc as plsc

N_USERS = 100000
N_ITEMS = 100000
D = 64
B = 4096
HIST = 50
HYPER_APPROX = 0.1
HYPER_PRIOR = 1.0
TAU = 4.0
BETA = 0.25

_NC = 2
_NS = 16
_NW = _NC * _NS
_BPW = B // _NW

_R = 128
_NT = B // _R
_HP = 56


def _sc_gather_body(uidx_h, iidx_h, hist_h, uemb_h, iemb_h,
                    refer_h, q_h, ie_h, kv_h,
                    idx_v, hist_v, emb_v, row_v, sem):
    wid = lax.axis_index("s") * _NC + lax.axis_index("c")
    base = wid * _BPW
    pltpu.sync_copy(uidx_h.at[pl.ds(base, _BPW)], idx_v)
    pltpu.async_copy(hist_h.at[idx_v], hist_v, sem).wait()
    pltpu.sync_copy(hist_v, refer_h.at[pl.ds(base, _BPW)])
    pltpu.async_copy(uemb_h.at[idx_v], emb_v, sem).wait()
    pltpu.sync_copy(emb_v, q_h.at[pl.ds(base, _BPW)])
    pltpu.sync_copy(iidx_h.at[pl.ds(base, _BPW)], idx_v)
    pltpu.async_copy(iemb_h.at[idx_v], emb_v, sem).wait()
    pltpu.sync_copy(emb_v, ie_h.at[pl.ds(base, _BPW)])

    def body(i, carry):
        pltpu.async_copy(iemb_h.at[hist_v.at[i, pl.ds(0, _HP)]], row_v,
                         sem).wait()
        pltpu.sync_copy(row_v, kv_h.at[base + i])
        return carry

    lax.fori_loop(0, _BPW, body, 0)


_sc_gather = functools.partial(
    pl.kernel,
    mesh=plsc.VectorSubcoreMesh(core_axis_name="c", subcore_axis_name="s"),
    compiler_params=pltpu.CompilerParams(use_tc_tiling_on_sc=False),
    out_type=[
        jax.ShapeDtypeStruct((B, D), jnp.int32),
        jax.ShapeDtypeStruct((B, D), jnp.float32),
        jax.ShapeDtypeStruct((B, D), jnp.float32),
        jax.ShapeDtypeStruct((B, _HP, D), jnp.float32),
    ],
    scratch_types=[
        pltpu.VMEM((_BPW,), jnp.int32),
        pltpu.VMEM((_BPW, D), jnp.int32),
        pltpu.VMEM((_BPW, D), jnp.float32),
        pltpu.VMEM((_HP, D), jnp.float32),
        pltpu.SemaphoreType.DMA,
    ],
)(_sc_gather_body)


def _tc_attend_body(refer_r, iidx_r, q_r, ie_r, kv_r, e1_r, e2_r,
                    W_r, bW_r, vrow_r, bv_r, gu_r, bu_r, gi_r, bi_r,
                    pW_r, pb_r, logit_r, kl_r, mk_r):
    refer = refer_r[...]
    iidx = iidx_r[...]
    valid = lax.broadcasted_iota(jnp.int32, (_R, _HP), 1) < HIST
    mask = jnp.logical_not((refer == iidx) | (refer == N_ITEMS)) & valid
    maskf = mask.astype(jnp.float32)
    q = q_r[...]
    kv = kv_r[...]
    W = W_r[...]
    qw = jnp.dot(q, W[:D], preferred_element_type=jnp.float32)
    kw = jnp.dot(kv.reshape(_R * _HP, D), W[D:],
                 preferred_element_type=jnp.float32).reshape(_R, _HP, D)
    h = jnp.tanh(kw + qw[:, None, :] + bW_r[...][None])
    s = (jnp.sum(h * vrow_r[...][None], axis=-1) + bv_r[...]) / TAU
    s = jnp.where(mask, s, jnp.float32(-1e9))
    m = jnp.max(s, axis=-1, keepdims=True)
    p = jnp.exp(s - m)
    phi = p / jnp.sum(p, axis=-1, keepdims=True)
    phi = phi * maskf
    mu = jnp.log(phi + 1e-10)

    def wavg(eps):
        w = jnp.exp(mu + HYPER_APPROX * eps) * maskf
        w = w / (jnp.sum(w, axis=-1, keepdims=True) + 1e-10)
        return jnp.sum(w[:, :, None] * kv, axis=1)

    out1 = wavg(e1_r[...])
    out2 = wavg(e2_r[...])

    lengths = jnp.sum(maskf, axis=-1, keepdims=True)
    mu_prior = -jnp.log(jnp.maximum(lengths, 1.0))
    kl_c = jnp.float32(jnp.log(HYPER_PRIOR / HYPER_APPROX) - 0.5)
    kl_el = kl_c + (HYPER_APPROX ** 2 + (mu - mu_prior) ** 2) / (2.0 * HYPER_PRIOR ** 2)
    klp = jnp.sum(kl_el * maskf)
    mkp = jnp.sum(maskf)

    def ln(x, g, b):
        mean = jnp.mean(x, axis=-1, keepdims=True)
        var = jnp.mean((x - mean) ** 2, axis=-1, keepdims=True)
        return (x - mean) / jnp.sqrt(var + 1e-5) * g + b

    ue = ln(out1 * q, gu_r[...], bu_r[...])
    ie = ln(out2 * ie_r[...], gi_r[...], bi_r[...])
    pred = ue * ie
    logit_r[...] = jnp.sum(pred * pW_r[...], axis=-1, keepdims=True) + pb_r[...]

    @pl.when(pl.program_id(0) == 0)
    def _():
        kl_r[...] = jnp.zeros((1, 1), jnp.float32)
        mk_r[...] = jnp.zeros((1, 1), jnp.float32)

    kl_r[...] += klp.reshape(1, 1)
    mk_r[...] += mkp.reshape(1, 1)


_tc_attend = pl.pallas_call(
    _tc_attend_body,
    grid=(_NT,),
    in_specs=[
        pl.BlockSpec((_R, _HP), lambda i: (i, 0)),
        pl.BlockSpec((_R, 1), lambda i: (i, 0)),
        pl.BlockSpec((_R, D), lambda i: (i, 0)),
        pl.BlockSpec((_R, D), lambda i: (i, 0)),
        pl.BlockSpec((_R, _HP, D), lambda i: (i, 0, 0)),
        pl.BlockSpec((_R, _HP), lambda i: (i, 0)),
        pl.BlockSpec((_R, _HP), lambda i: (i, 0)),
        pl.BlockSpec((2 * D, D), lambda i: (0, 0)),
        pl.BlockSpec((1, D), lambda i: (0, 0)),
        pl.BlockSpec((1, D), lambda i: (0, 0)),
        pl.BlockSpec((1, 1), lambda i: (0, 0)),
        pl.BlockSpec((1, D), lambda i: (0, 0)),
        pl.BlockSpec((1, D), lambda i: (0, 0)),
        pl.BlockSpec((1, D), lambda i: (0, 0)),
        pl.BlockSpec((1, D), lambda i: (0, 0)),
        pl.BlockSpec((1, D), lambda i: (0, 0)),
        pl.BlockSpec((1, 1), lambda i: (0, 0)),
    ],
    out_specs=[
        pl.BlockSpec((_R, 1), lambda i: (i, 0)),
        pl.BlockSpec((1, 1), lambda i: (0, 0)),
        pl.BlockSpec((1, 1), lambda i: (0, 0)),
    ],
    out_shape=[
        jax.ShapeDtypeStruct((B, 1), jnp.float32),
        jax.ShapeDtypeStruct((1, 1), jnp.float32),
        jax.ShapeDtypeStruct((1, 1), jnp.float32),
    ],
)


def kernel(user_idx, item_idx, user_hist, item_hist, user_embed_w, item_embed_w,
           W_u, bW_u, v_u, bv_u, norm_u_g, norm_u_b, norm_i_g, norm_i_b,
           pred_W, pred_b):
    del item_hist
    uidx = user_idx.astype(jnp.int32)
    iidx = item_idx.astype(jnp.int32)
    hist = jnp.pad(user_hist.astype(jnp.int32), ((0, 0), (0, D - HIST)))
    refer64, q, ie, kv = _sc_gather(uidx, iidx, hist, user_embed_w,
                                    item_embed_w)
    refer = refer64[:, :_HP]
    eps1 = jax.random.normal(jax.random.key(42), (B, HIST), dtype=jnp.float32)
    eps2 = jax.random.normal(jax.random.key(43), (B, HIST), dtype=jnp.float32)
    pad = ((0, 0), (0, _HP - HIST))
    logit2, kls, mks = _tc_attend(
        refer, iidx.reshape(B, 1), q, ie, kv,
        jnp.pad(eps1, pad), jnp.pad(eps2, pad),
        W_u, bW_u.reshape(1, D), v_u.reshape(1, D), bv_u.reshape(1, 1),
        norm_u_g.reshape(1, D), norm_u_b.reshape(1, D),
        norm_i_g.reshape(1, D), norm_i_b.reshape(1, D),
        pred_W.reshape(1, D), pred_b.reshape(1, 1))
    kl = BETA * kls[0, 0] / jnp.maximum(mks[0, 0], 1.0)
    return logit2[:, 0], kl


# speedup vs baseline: 2.6893x; 1.0014x over previous
import functools

import jax
import jax.numpy as jnp
from jax import lax
from jax.experimental import pallas as pl
from jax.experimental.pallas import tpu as pltpu
from jax.experimental.pallas import tpu_sc as plsc

N_USERS = 100000
N_ITEMS = 100000
D = 64
B = 4096
HIST = 50
HYPER_APPROX = 0.1
HYPER_PRIOR = 1.0
TAU = 4.0
BETA = 0.25

_NC = 2
_NS = 16
_NW = _NC * _NS
_BPW = B // _NW
_GB = 8

_R = 128
_NT = B // _R
_HP = 56


def _sc_gather_body(uidx_h, iidx_h, hist_h, uemb_h, iemb_h,
                    refer_h, q_h, ie_h, kv_h,
                    idx_v, hist_v, emb_v, rows_v, sem, wsem):
    wid = lax.axis_index("s") * _NC + lax.axis_index("c")
    base = wid * _BPW
    pltpu.sync_copy(uidx_h.at[pl.ds(base, _BPW)], idx_v)
    pltpu.async_copy(hist_h.at[idx_v], hist_v, sem).wait()
    pltpu.sync_copy(hist_v, refer_h.at[pl.ds(base, _BPW)])
    pltpu.async_copy(uemb_h.at[idx_v], emb_v, sem).wait()
    pltpu.sync_copy(emb_v, q_h.at[pl.ds(base, _BPW)])
    pltpu.sync_copy(iidx_h.at[pl.ds(base, _BPW)], idx_v)
    pltpu.async_copy(iemb_h.at[idx_v], emb_v, sem).wait()
    pltpu.sync_copy(emb_v, ie_h.at[pl.ds(base, _BPW)])

    def body(g, carry):
        s = lax.rem(g, 2)
        i0 = g * _GB

        @pl.when(g >= 2)
        def _():
            pltpu.make_async_copy(rows_v.at[s], kv_h.at[pl.ds(base, _GB)],
                                  wsem).wait()

        cps = [pltpu.async_copy(
                   iemb_h.at[hist_v.at[i0 + j, pl.ds(0, _HP)]],
                   rows_v.at[s, j], sem) for j in range(_GB)]
        for cp in cps:
            cp.wait()
        pltpu.async_copy(rows_v.at[s], kv_h.at[pl.ds(base + i0, _GB)], wsem)
        return carry

    ngrp = _BPW // _GB
    lax.fori_loop(0, ngrp, body, 0)
    pltpu.make_async_copy(rows_v.at[0], kv_h.at[pl.ds(base, _GB)], wsem).wait()
    pltpu.make_async_copy(rows_v.at[1], kv_h.at[pl.ds(base, _GB)], wsem).wait()


_sc_gather = functools.partial(
    pl.kernel,
    mesh=plsc.VectorSubcoreMesh(core_axis_name="c", subcore_axis_name="s"),
    compiler_params=pltpu.CompilerParams(use_tc_tiling_on_sc=False),
    out_type=[
        jax.ShapeDtypeStruct((B, D), jnp.int32),
        jax.ShapeDtypeStruct((B, D), jnp.float32),
        jax.ShapeDtypeStruct((B, D), jnp.float32),
        jax.ShapeDtypeStruct((B, _HP, D), jnp.float32),
    ],
    scratch_types=[
        pltpu.VMEM((_BPW,), jnp.int32),
        pltpu.VMEM((_BPW, D), jnp.int32),
        pltpu.VMEM((_BPW, D), jnp.float32),
        pltpu.VMEM((2, _GB, _HP, D), jnp.float32),
        pltpu.SemaphoreType.DMA,
        pltpu.SemaphoreType.DMA,
    ],
)(_sc_gather_body)


def _tc_attend_body(refer_r, iidx_r, q_r, ie_r, kv_r, e1_r, e2_r,
                    W_r, bW_r, vrow_r, bv_r, gu_r, bu_r, gi_r, bi_r,
                    pW_r, pb_r, logit_r, kl_r, mk_r):
    refer = refer_r[...]
    iidx = iidx_r[...]
    valid = lax.broadcasted_iota(jnp.int32, (_R, _HP), 1) < HIST
    mask = jnp.logical_not((refer == iidx) | (refer == N_ITEMS)) & valid
    maskf = mask.astype(jnp.float32)
    q = q_r[...]
    kv = kv_r[...]
    W = W_r[...]
    qw = jnp.dot(q, W[:D], preferred_element_type=jnp.float32)
    kw = jnp.dot(kv.reshape(_R * _HP, D), W[D:],
                 preferred_element_type=jnp.float32).reshape(_R, _HP, D)
    h = jnp.tanh(kw + qw[:, None, :] + bW_r[...][None])
    s = (jnp.sum(h * vrow_r[...][None], axis=-1) + bv_r[...]) / TAU
    s = jnp.where(mask, s, jnp.float32(-1e9))
    m = jnp.max(s, axis=-1, keepdims=True)
    p = jnp.exp(s - m)
    phi = p / jnp.sum(p, axis=-1, keepdims=True)
    phi = phi * maskf
    mu = jnp.log(phi + 1e-10)

    def wavg(eps):
        w = jnp.exp(mu + HYPER_APPROX * eps) * maskf
        w = w / (jnp.sum(w, axis=-1, keepdims=True) + 1e-10)
        return jnp.sum(w[:, :, None] * kv, axis=1)

    out1 = wavg(e1_r[...])
    out2 = wavg(e2_r[...])

    lengths = jnp.sum(maskf, axis=-1, keepdims=True)
    mu_prior = -jnp.log(jnp.maximum(lengths, 1.0))
    kl_c = jnp.float32(jnp.log(HYPER_PRIOR / HYPER_APPROX) - 0.5)
    kl_el = kl_c + (HYPER_APPROX ** 2 + (mu - mu_prior) ** 2) / (2.0 * HYPER_PRIOR ** 2)
    klp = jnp.sum(kl_el * maskf)
    mkp = jnp.sum(maskf)

    def ln(x, g, b):
        mean = jnp.mean(x, axis=-1, keepdims=True)
        var = jnp.mean((x - mean) ** 2, axis=-1, keepdims=True)
        return (x - mean) / jnp.sqrt(var + 1e-5) * g + b

    ue = ln(out1 * q, gu_r[...], bu_r[...])
    ie = ln(out2 * ie_r[...], gi_r[...], bi_r[...])
    pred = ue * ie
    logit_r[...] = jnp.sum(pred * pW_r[...], axis=-1, keepdims=True) + pb_r[...]

    @pl.when(pl.program_id(0) == 0)
    def _():
        kl_r[...] = jnp.zeros((1, 1), jnp.float32)
        mk_r[...] = jnp.zeros((1, 1), jnp.float32)

    kl_r[...] += klp.reshape(1, 1)
    mk_r[...] += mkp.reshape(1, 1)


_tc_attend = pl.pallas_call(
    _tc_attend_body,
    grid=(_NT,),
    in_specs=[
        pl.BlockSpec((_R, _HP), lambda i: (i, 0)),
        pl.BlockSpec((_R, 1), lambda i: (i, 0)),
        pl.BlockSpec((_R, D), lambda i: (i, 0)),
        pl.BlockSpec((_R, D), lambda i: (i, 0)),
        pl.BlockSpec((_R, _HP, D), lambda i: (i, 0, 0)),
        pl.BlockSpec((_R, _HP), lambda i: (i, 0)),
        pl.BlockSpec((_R, _HP), lambda i: (i, 0)),
        pl.BlockSpec((2 * D, D), lambda i: (0, 0)),
        pl.BlockSpec((1, D), lambda i: (0, 0)),
        pl.BlockSpec((1, D), lambda i: (0, 0)),
        pl.BlockSpec((1, 1), lambda i: (0, 0)),
        pl.BlockSpec((1, D), lambda i: (0, 0)),
        pl.BlockSpec((1, D), lambda i: (0, 0)),
        pl.BlockSpec((1, D), lambda i: (0, 0)),
        pl.BlockSpec((1, D), lambda i: (0, 0)),
        pl.BlockSpec((1, D), lambda i: (0, 0)),
        pl.BlockSpec((1, 1), lambda i: (0, 0)),
    ],
    out_specs=[
        pl.BlockSpec((_R, 1), lambda i: (i, 0)),
        pl.BlockSpec((1, 1), lambda i: (0, 0)),
        pl.BlockSpec((1, 1), lambda i: (0, 0)),
    ],
    out_shape=[
        jax.ShapeDtypeStruct((B, 1), jnp.float32),
        jax.ShapeDtypeStruct((1, 1), jnp.float32),
        jax.ShapeDtypeStruct((1, 1), jnp.float32),
    ],
)


def kernel(user_idx, item_idx, user_hist, item_hist, user_embed_w, item_embed_w,
           W_u, bW_u, v_u, bv_u, norm_u_g, norm_u_b, norm_i_g, norm_i_b,
           pred_W, pred_b):
    del item_hist
    uidx = user_idx.astype(jnp.int32)
    iidx = item_idx.astype(jnp.int32)
    hist = jnp.pad(user_hist.astype(jnp.int32), ((0, 0), (0, D - HIST)))
    refer64, q, ie, kv = _sc_gather(uidx, iidx, hist, user_embed_w,
                                    item_embed_w)
    refer = refer64[:, :_HP]
    eps1 = jax.random.normal(jax.random.key(42), (B, HIST), dtype=jnp.float32)
    eps2 = jax.random.normal(jax.random.key(43), (B, HIST), dtype=jnp.float32)
    pad = ((0, 0), (0, _HP - HIST))
    logit2, kls, mks = _tc_attend(
        refer, iidx.reshape(B, 1), q, ie, kv,
        jnp.pad(eps1, pad), jnp.pad(eps2, pad),
        W_u, bW_u.reshape(1, D), v_u.reshape(1, D), bv_u.reshape(1, 1),
        norm_u_g.reshape(1, D), norm_u_b.reshape(1, D),
        norm_i_g.reshape(1, D), norm_i_b.reshape(1, D),
        pred_W.reshape(1, D), pred_b.reshape(1, 1))
    kl = BETA * kls[0, 0] / jnp.maximum(mks[0, 0], 1.0)
    return logit2[:, 0], kl
